# Initial kernel scaffold; baseline (speedup 1.0000x reference)
import jax
import jax.numpy as jnp
from jax.experimental import pallas as pl


def kernel(x, src0, dst0, src1, dst1, nid0, nid1, hist0, hist1, W0, b0, gamma, beta, W1, b1, batch_size):
    raise NotImplementedError("write your pallas kernel here")


# speedup vs baseline: 1.5043x; 1.5043x over previous
import functools

import jax
import jax.numpy as jnp
from jax import lax
from jax.experimental import pallas as pl
from jax.experimental.pallas import tpu as pltpu
from jax.experimental.pallas import tpu_sc as plsc

N0 = 32768
N1 = 8192
B = 1024
E0 = 262144
E1 = 32768
D = 256
DHALF = 128
NCLS = 64
NSUB = 16
NCORE = 2

P0_ROWS = N0 - B
P1_ROWS = N1 - B
P0_PER_W = P0_ROWS // (NSUB * NCORE)
P1_PER_W = P1_ROWS // (NSUB * NCORE)
EC = 128


def _sc_mesh():
    return plsc.VectorSubcoreMesh(core_axis_name="c", subcore_axis_name="s")


def _match_body(pull_ref, push_ref, jraw_ref, jclip_ref):
    pull = pull_ref[0, 0, :]
    push = push_ref[0, :]
    eq = pull[:, None] == push[None, :]
    jj = lax.broadcasted_iota(jnp.int32, (B, B), 1)
    j = jnp.max(jnp.where(eq, jj, -1), axis=1)
    jraw_ref[0, 0, :] = j
    jclip_ref[0, 0, :] = jnp.maximum(j, 0)


def _match_maps(nid_tail3, push2, nblk):
    return pl.pallas_call(
        _match_body,
        grid=(nblk,),
        in_specs=[pl.BlockSpec((1, 1, B), lambda i: (i, 0, 0)),
                  pl.BlockSpec((1, B), lambda i: (0, 0))],
        out_specs=[pl.BlockSpec((1, 1, B), lambda i: (i, 0, 0)),
                   pl.BlockSpec((1, 1, B), lambda i: (i, 0, 0))],
        out_shape=[jax.ShapeDtypeStruct((nblk, 1, B), jnp.int32),
                   jax.ShapeDtypeStruct((nblk, 1, B), jnp.int32)],
    )(nid_tail3, push2)


def _pulls_counts_body(hist0, x, hist1, nid0t, j0c, nid1t,
                       src0, dst0, src1, dst1, ones_h, zeros_h,
                       p0, o0, p1, c_out0, c_in0, c_out1, c_in1,
                       idx_v, rows_v, ones_v, cbuf_v, ea_v, eb_v,
                       cnta_sh, cntb_sh):
    c = lax.axis_index("c")
    s = lax.axis_index("s")
    wid = s * NCORE + c

    def gather_rows(table, idxs_hbm, n_per_w, out):
        base = wid * n_per_w
        pltpu.sync_copy(idxs_hbm.at[pl.ds(base, n_per_w)],
                        idx_v.at[pl.ds(0, n_per_w)])
        off = 0
        while off < n_per_w:
            chunk = min(128, n_per_w - off)
            rows = rows_v.at[pl.ds(0, chunk)]
            pltpu.sync_copy(table.at[idx_v.at[pl.ds(off, chunk)]], rows)
            pltpu.sync_copy(rows, out.at[pl.ds(base + off, chunk)])
            off += chunk

    gather_rows(hist0, nid0t, P0_PER_W, p0)
    gather_rows(x, j0c, P0_PER_W, o0)
    gather_rows(hist1, nid1t, P1_PER_W, p1)

    pltpu.sync_copy(ones_h, ones_v)
    pltpu.sync_copy(zeros_h, cbuf_v)

    @pl.when(c == 0)
    def _():
        pltpu.sync_copy(cbuf_v, cnta_sh.at[pl.ds(s * 2048, 2048)])
        pltpu.sync_copy(cbuf_v.at[pl.ds(0, 512)],
                        cntb_sh.at[pl.ds(s * 512, 512)])

    @pl.when(c == 1)
    def _():
        pltpu.sync_copy(cbuf_v.at[pl.ds(0, 512)],
                        cnta_sh.at[pl.ds(s * 512, 512)])
        pltpu.sync_copy(cbuf_v.at[pl.ds(0, 64)],
                        cntb_sh.at[pl.ds(s * 64, 64)])

    plsc.subcore_barrier()

    def count_edges(src, dst, e_per_w):
        def body(k, carry):
            base = s * e_per_w + k * EC
            pltpu.sync_copy(src.at[pl.ds(base, EC)], ea_v)
            pltpu.sync_copy(ones_v, cnta_sh.at[ea_v], add=True)
            pltpu.sync_copy(dst.at[pl.ds(base, EC)], eb_v)
            pltpu.sync_copy(ones_v, cntb_sh.at[eb_v], add=True)
            return carry
        lax.fori_loop(0, e_per_w // EC, body, 0)

    @pl.when(c == 0)
    def _():
        count_edges(src0, dst0, E0 // NSUB)

    @pl.when(c == 1)
    def _():
        count_edges(src1, dst1, E1 // NSUB)

    plsc.subcore_barrier()

    def spmem_to_hbm(sh, n, out):
        pltpu.sync_copy(sh.at[pl.ds(s * n, n)], cbuf_v.at[pl.ds(0, n)])
        pltpu.sync_copy(cbuf_v.at[pl.ds(0, n)], out.at[pl.ds(s * n, n)])

    @pl.when(c == 0)
    def _():
        spmem_to_hbm(cnta_sh, 2048, c_out0)
        spmem_to_hbm(cntb_sh, 512, c_in0)

    @pl.when(c == 1)
    def _():
        spmem_to_hbm(cnta_sh, 512, c_out1)
        spmem_to_hbm(cntb_sh, 64, c_in1)


def _pulls_counts(hist0, x, hist1, nid0t, j0c, nid1t, src0, dst0, src1, dst1):
    ones_h = jnp.ones((EC,), jnp.float32)
    zeros_h = jnp.zeros((2048,), jnp.float32)
    f = pl.kernel(
        _pulls_counts_body,
        out_type=[jax.ShapeDtypeStruct((P0_ROWS, D), jnp.float32),
                  jax.ShapeDtypeStruct((P0_ROWS, D), jnp.float32),
                  jax.ShapeDtypeStruct((P1_ROWS, D), jnp.float32),
                  jax.ShapeDtypeStruct((N0,), jnp.float32),
                  jax.ShapeDtypeStruct((N1,), jnp.float32),
                  jax.ShapeDtypeStruct((N1,), jnp.float32),
                  jax.ShapeDtypeStruct((B,), jnp.float32)],
        mesh=_sc_mesh(),
        scratch_types=[pltpu.VMEM((P0_PER_W,), jnp.int32),
                       pltpu.VMEM((128, D), jnp.float32),
                       pltpu.VMEM((EC,), jnp.float32),
                       pltpu.VMEM((2048,), jnp.float32),
                       pltpu.VMEM((EC,), jnp.int32),
                       pltpu.VMEM((EC,), jnp.int32),
                       pltpu.VMEM_SHARED((N0,), jnp.float32),
                       pltpu.VMEM_SHARED((N1,), jnp.float32)],
    )
    return f(hist0, x, hist1, nid0t, j0c, nid1t, src0, dst0, src1, dst1,
             ones_h, zeros_h)


def _xs_body(xh_ref, p_ref, o_ref, j_ref, cnt_ref, lo_ref, hi_ref):
    i = pl.program_id(0)
    rs = lax.rsqrt(jnp.maximum(cnt_ref[0], 1.0))
    j = j_ref[0]
    tail = jnp.where(j >= 0, o_ref[0], p_ref[0])
    rows = jnp.where(i == 0, xh_ref[0], tail)
    scaled = rows * rs
    lo_ref[0] = scaled[:, :DHALF]
    hi_ref[0] = scaled[:, DHALF:]


def _assemble_xs(xh3, p03, o03, j03, cnt3):
    nblk = N0 // B
    back = lambda i: (jnp.maximum(i - 1, 0), 0, 0)
    lo, hi = pl.pallas_call(
        _xs_body,
        grid=(nblk,),
        in_specs=[pl.BlockSpec((1, B, D), lambda i: (0, 0, 0)),
                  pl.BlockSpec((1, B, D), back),
                  pl.BlockSpec((1, B, D), back),
                  pl.BlockSpec((1, B, 1), back),
                  pl.BlockSpec((1, B, 1), lambda i: (i, 0, 0))],
        out_specs=[pl.BlockSpec((1, B, DHALF), lambda i: (i, 0, 0)),
                   pl.BlockSpec((1, B, DHALF), lambda i: (i, 0, 0))],
        out_shape=[jax.ShapeDtypeStruct((nblk, B, DHALF), jnp.float32),
                   jax.ShapeDtypeStruct((nblk, B, DHALF), jnp.float32)],
    )(xh3, p03, o03, j03, cnt3)
    return lo.reshape(N0, DHALF), hi.reshape(N0, DHALF)


def _conv_body(n_dst, e_per_w, lo_t, hi_t, src, dst, zeros_h,
               agg_lo, agg_hi, eidx_v, didx_v, rows_v, acc_sh):
    c = lax.axis_index("c")
    s = lax.axis_index("s")
    r_per_w = n_dst // NSUB
    rchunk = min(r_per_w, EC)

    pltpu.sync_copy(zeros_h.at[pl.ds(0, rchunk)], rows_v.at[pl.ds(0, rchunk)])
    for r in range(r_per_w // rchunk):
        pltpu.sync_copy(rows_v.at[pl.ds(0, rchunk)],
                        acc_sh.at[pl.ds(s * r_per_w + r * rchunk, rchunk)])
    plsc.subcore_barrier()

    def body(k, carry):
        base = s * e_per_w + k * EC
        pltpu.sync_copy(src.at[pl.ds(base, EC)], eidx_v)
        pltpu.sync_copy(dst.at[pl.ds(base, EC)], didx_v)

        @pl.when(c == 0)
        def _():
            pltpu.sync_copy(lo_t.at[eidx_v], rows_v)

        @pl.when(c == 1)
        def _():
            pltpu.sync_copy(hi_t.at[eidx_v], rows_v)

        pltpu.sync_copy(rows_v, acc_sh.at[didx_v], add=True)
        return carry

    lax.fori_loop(0, e_per_w // EC, body, 0)
    plsc.subcore_barrier()

    def copy_out(agg):
        for r in range(r_per_w // rchunk):
            off = s * r_per_w + r * rchunk
            pltpu.sync_copy(acc_sh.at[pl.ds(off, rchunk)],
                            rows_v.at[pl.ds(0, rchunk)])
            pltpu.sync_copy(rows_v.at[pl.ds(0, rchunk)],
                            agg.at[pl.ds(off, rchunk)])

    @pl.when(c == 0)
    def _():
        copy_out(agg_lo)

    @pl.when(c == 1)
    def _():
        copy_out(agg_hi)


def _edge_conv(lo_t, hi_t, src, dst, n_dst, n_edges):
    e_per_w = n_edges // NSUB
    zeros_h = jnp.zeros((EC, DHALF), jnp.float32)
    f = pl.kernel(
        functools.partial(_conv_body, n_dst, e_per_w),
        out_type=[jax.ShapeDtypeStruct((n_dst, DHALF), jnp.float32),
                  jax.ShapeDtypeStruct((n_dst, DHALF), jnp.float32)],
        mesh=_sc_mesh(),
        scratch_types=[pltpu.VMEM((EC,), jnp.int32),
                       pltpu.VMEM((EC,), jnp.int32),
                       pltpu.VMEM((EC, DHALF), jnp.float32),
                       pltpu.VMEM_SHARED((n_dst, DHALF), jnp.float32)],
    )
    return f(lo_t, hi_t, src, dst, zeros_h)


def _mid_body(alo_ref, ahi_ref, cin_ref, w_ref, b_ref, g_ref, be_ref,
              p1_ref, j1_ref, cout_ref, lo_ref, hi_ref):
    rdi = lax.rsqrt(jnp.maximum(cin_ref[...], 1.0))
    h = (jnp.dot(alo_ref[...] * rdi, w_ref[:DHALF, :],
                 preferred_element_type=jnp.float32)
         + jnp.dot(ahi_ref[...] * rdi, w_ref[DHALF:, :],
                   preferred_element_type=jnp.float32)
         + b_ref[...])
    mean = jnp.mean(h, axis=0, keepdims=True)
    var = jnp.mean((h - mean) ** 2, axis=0, keepdims=True)
    hn = (h - mean) * lax.rsqrt(var + 1e-5) * g_ref[...] + be_ref[...]
    hr = jnp.maximum(hn, 0.0)
    head = hr[:B, :]

    parts = [head]
    jj = lax.broadcasted_iota(jnp.int32, (B, B), 1)
    for blk in range(P1_ROWS // B):
        jb = j1_ref[blk]
        oh = (jb == jj).astype(jnp.float32)
        o1 = jnp.dot(oh, head, preferred_element_type=jnp.float32)
        pb = p1_ref[blk * B:(blk + 1) * B, :]
        parts.append(jnp.where(jb >= 0, o1, pb))
    full = jnp.concatenate(parts, axis=0)
    hs = full * lax.rsqrt(jnp.maximum(cout_ref[...], 1.0))
    lo_ref[...] = hs[:, :DHALF]
    hi_ref[...] = hs[:, DHALF:]


def _mid_layer(agg_lo, agg_hi, cin0, W0, b0, gamma, beta, p1, j13, cout1):
    return pl.pallas_call(
        _mid_body,
        out_shape=[jax.ShapeDtypeStruct((N1, DHALF), jnp.float32),
                   jax.ShapeDtypeStruct((N1, DHALF), jnp.float32)],
    )(agg_lo, agg_hi, cin0.reshape(N1, 1), W0, b0.reshape(1, D),
      gamma.reshape(1, D), beta.reshape(1, D), p1, j13,
      cout1.reshape(N1, 1))


def _out_body(alo_ref, ahi_ref, cin_ref, w_ref, b_ref, out_ref):
    rdi = lax.rsqrt(jnp.maximum(cin_ref[...], 1.0))
    z = (jnp.dot(alo_ref[...] * rdi, w_ref[:DHALF, :],
                 preferred_element_type=jnp.float32)
         + jnp.dot(ahi_ref[...] * rdi, w_ref[DHALF:, :],
                   preferred_element_type=jnp.float32)
         + b_ref[...])
    m = jnp.max(z, axis=1, keepdims=True)
    lse = m + jnp.log(jnp.sum(jnp.exp(z - m), axis=1, keepdims=True))
    out_ref[...] = z - lse


def _out_layer(agg_lo, agg_hi, cin1, W1, b1):
    return pl.pallas_call(
        _out_body,
        out_shape=jax.ShapeDtypeStruct((B, NCLS), jnp.float32),
    )(agg_lo, agg_hi, cin1.reshape(B, 1), W1, b1.reshape(1, NCLS))


def kernel(x, src0, dst0, src1, dst1, nid0, nid1, hist0, hist1,
           W0, b0, gamma, beta, W1, b1, batch_size):
    del batch_size
    i32 = jnp.int32
    src0, dst0 = src0.astype(i32), dst0.astype(i32)
    src1, dst1 = src1.astype(i32), dst1.astype(i32)
    nid0, nid1 = nid0.astype(i32), nid1.astype(i32)

    j0raw3, j0clip3 = _match_maps(nid0[B:].reshape(-1, 1, B),
                                  nid0[:B].reshape(1, B), P0_ROWS // B)
    j1raw3, j1clip3 = _match_maps(nid1[B:].reshape(-1, 1, B),
                                  nid1[:B].reshape(1, B), P1_ROWS // B)

    p0, o0, p1, c_out0, c_in0, c_out1, c_in1 = _pulls_counts(
        hist0, x, hist1, nid0[B:], j0clip3.reshape(-1), nid1[B:],
        src0, dst0, src1, dst1)

    xs_lo, xs_hi = _assemble_xs(
        x[:B].reshape(1, B, D), p0.reshape(-1, B, D), o0.reshape(-1, B, D),
        j0raw3.reshape(-1, B, 1), c_out0.reshape(-1, B, 1))

    agg_lo, agg_hi = _edge_conv(xs_lo, xs_hi, src0, dst0, N1, E0)

    hs_lo, hs_hi = _mid_layer(agg_lo, agg_hi, c_in0, W0, b0, gamma, beta,
                              p1, j1raw3.reshape(-1, B, 1), c_out1)

    agg1_lo, agg1_hi = _edge_conv(hs_lo, hs_hi, src1, dst1, B, E1)

    return _out_layer(agg1_lo, agg1_hi, c_in1, W1, b1)


# speedup vs baseline: 1.7121x; 1.1381x over previous
import functools

import jax
import jax.numpy as jnp
from jax import lax
from jax.experimental import pallas as pl
from jax.experimental.pallas import tpu as pltpu
from jax.experimental.pallas import tpu_sc as plsc

N0 = 32768
N1 = 8192
B = 1024
E0 = 262144
E1 = 32768
D = 256
DHALF = 128
NCLS = 64
NSUB = 16
NCORE = 2

P0_ROWS = N0 - B
P1_ROWS = N1 - B
P0_PER_W = P0_ROWS // (NSUB * NCORE)
P1_PER_W = P1_ROWS // (NSUB * NCORE)
EC = 128


def _sc_mesh():
    return plsc.VectorSubcoreMesh(core_axis_name="c", subcore_axis_name="s")


def _match_body(pull_ref, push_ref, jraw_ref, jclip_ref):
    pull = pull_ref[0, 0, :]
    push = push_ref[0, :]
    eq = pull[:, None] == push[None, :]
    jj = lax.broadcasted_iota(jnp.int32, (B, B), 1)
    j = jnp.max(jnp.where(eq, jj, -1), axis=1)
    jraw_ref[0, 0, :] = j
    jclip_ref[0, 0, :] = jnp.maximum(j, 0)


def _match_maps(nid_tail3, push2, nblk):
    return pl.pallas_call(
        _match_body,
        grid=(nblk,),
        in_specs=[pl.BlockSpec((1, 1, B), lambda i: (i, 0, 0)),
                  pl.BlockSpec((1, B), lambda i: (0, 0))],
        out_specs=[pl.BlockSpec((1, 1, B), lambda i: (i, 0, 0)),
                   pl.BlockSpec((1, 1, B), lambda i: (i, 0, 0))],
        out_shape=[jax.ShapeDtypeStruct((nblk, 1, B), jnp.int32),
                   jax.ShapeDtypeStruct((nblk, 1, B), jnp.int32)],
    )(nid_tail3, push2)


def _pulls_counts_body(hist0, x, hist1, nid0t, j0c, nid1t,
                       src0, dst0, src1, dst1, ones_h, zeros_h,
                       p0, o0, p1, c_out0, c_in0, c_out1, c_in1,
                       idx_v, rows_v, ones_v, cbuf_v, ea_v, eb_v,
                       cnta_sh, cntb_sh):
    c = lax.axis_index("c")
    s = lax.axis_index("s")
    wid = s * NCORE + c

    def gather_rows(table, idxs_hbm, n_per_w, out):
        base = wid * n_per_w
        pltpu.sync_copy(idxs_hbm.at[pl.ds(base, n_per_w)],
                        idx_v.at[pl.ds(0, n_per_w)])
        off = 0
        while off < n_per_w:
            chunk = min(128, n_per_w - off)
            rows = rows_v.at[pl.ds(0, chunk)]
            pltpu.sync_copy(table.at[idx_v.at[pl.ds(off, chunk)]], rows)
            pltpu.sync_copy(rows, out.at[pl.ds(base + off, chunk)])
            off += chunk

    gather_rows(hist0, nid0t, P0_PER_W, p0)
    gather_rows(x, j0c, P0_PER_W, o0)
    gather_rows(hist1, nid1t, P1_PER_W, p1)

    pltpu.sync_copy(ones_h, ones_v)
    pltpu.sync_copy(zeros_h, cbuf_v)

    @pl.when(c == 0)
    def _():
        pltpu.sync_copy(cbuf_v, cnta_sh.at[pl.ds(s * 2048, 2048)])
        pltpu.sync_copy(cbuf_v.at[pl.ds(0, 512)],
                        cntb_sh.at[pl.ds(s * 512, 512)])

    @pl.when(c == 1)
    def _():
        pltpu.sync_copy(cbuf_v.at[pl.ds(0, 512)],
                        cnta_sh.at[pl.ds(s * 512, 512)])
        pltpu.sync_copy(cbuf_v.at[pl.ds(0, 64)],
                        cntb_sh.at[pl.ds(s * 64, 64)])

    plsc.subcore_barrier()

    def count_edges(src, dst, e_per_w):
        def body(k, carry):
            base = s * e_per_w + k * EC
            pltpu.sync_copy(src.at[pl.ds(base, EC)], ea_v)
            pltpu.sync_copy(ones_v, cnta_sh.at[ea_v], add=True)
            pltpu.sync_copy(dst.at[pl.ds(base, EC)], eb_v)
            pltpu.sync_copy(ones_v, cntb_sh.at[eb_v], add=True)
            return carry
        lax.fori_loop(0, e_per_w // EC, body, 0)

    @pl.when(c == 0)
    def _():
        count_edges(src0, dst0, E0 // NSUB)

    @pl.when(c == 1)
    def _():
        count_edges(src1, dst1, E1 // NSUB)

    plsc.subcore_barrier()

    def spmem_to_hbm(sh, n, out):
        pltpu.sync_copy(sh.at[pl.ds(s * n, n)], cbuf_v.at[pl.ds(0, n)])
        pltpu.sync_copy(cbuf_v.at[pl.ds(0, n)], out.at[pl.ds(s * n, n)])

    @pl.when(c == 0)
    def _():
        spmem_to_hbm(cnta_sh, 2048, c_out0)
        spmem_to_hbm(cntb_sh, 512, c_in0)

    @pl.when(c == 1)
    def _():
        spmem_to_hbm(cnta_sh, 512, c_out1)
        spmem_to_hbm(cntb_sh, 64, c_in1)


def _pulls_counts(hist0, x, hist1, nid0t, j0c, nid1t, src0, dst0, src1, dst1):
    ones_h = jnp.ones((EC,), jnp.float32)
    zeros_h = jnp.zeros((2048,), jnp.float32)
    f = pl.kernel(
        _pulls_counts_body,
        out_type=[jax.ShapeDtypeStruct((P0_ROWS, D), jnp.float32),
                  jax.ShapeDtypeStruct((P0_ROWS, D), jnp.float32),
                  jax.ShapeDtypeStruct((P1_ROWS, D), jnp.float32),
                  jax.ShapeDtypeStruct((N0,), jnp.float32),
                  jax.ShapeDtypeStruct((N1,), jnp.float32),
                  jax.ShapeDtypeStruct((N1,), jnp.float32),
                  jax.ShapeDtypeStruct((B,), jnp.float32)],
        mesh=_sc_mesh(),
        scratch_types=[pltpu.VMEM((P0_PER_W,), jnp.int32),
                       pltpu.VMEM((128, D), jnp.float32),
                       pltpu.VMEM((EC,), jnp.float32),
                       pltpu.VMEM((2048,), jnp.float32),
                       pltpu.VMEM((EC,), jnp.int32),
                       pltpu.VMEM((EC,), jnp.int32),
                       pltpu.VMEM_SHARED((N0,), jnp.float32),
                       pltpu.VMEM_SHARED((N1,), jnp.float32)],
    )
    return f(hist0, x, hist1, nid0t, j0c, nid1t, src0, dst0, src1, dst1,
             ones_h, zeros_h)


def _xs_body(xh_ref, p_ref, o_ref, j_ref, cnt_ref, lo_ref, hi_ref):
    i = pl.program_id(0)
    rs = lax.rsqrt(jnp.maximum(cnt_ref[0], 1.0))
    j = j_ref[0]
    tail = jnp.where(j >= 0, o_ref[0], p_ref[0])
    rows = jnp.where(i == 0, xh_ref[0], tail)
    scaled = rows * rs
    lo_ref[0] = scaled[:, :DHALF]
    hi_ref[0] = scaled[:, DHALF:]


def _assemble_xs(xh3, p03, o03, j03, cnt3):
    nblk = N0 // B
    back = lambda i: (jnp.maximum(i - 1, 0), 0, 0)
    lo, hi = pl.pallas_call(
        _xs_body,
        grid=(nblk,),
        in_specs=[pl.BlockSpec((1, B, D), lambda i: (0, 0, 0)),
                  pl.BlockSpec((1, B, D), back),
                  pl.BlockSpec((1, B, D), back),
                  pl.BlockSpec((1, B, 1), back),
                  pl.BlockSpec((1, B, 1), lambda i: (i, 0, 0))],
        out_specs=[pl.BlockSpec((1, B, DHALF), lambda i: (i, 0, 0)),
                   pl.BlockSpec((1, B, DHALF), lambda i: (i, 0, 0))],
        out_shape=[jax.ShapeDtypeStruct((nblk, B, DHALF), jnp.float32),
                   jax.ShapeDtypeStruct((nblk, B, DHALF), jnp.float32)],
    )(xh3, p03, o03, j03, cnt3)
    return lo.reshape(N0, DHALF), hi.reshape(N0, DHALF)


def _conv_body(n_dst, nch, lo_t, hi_t, src2, dst2, zeros_h,
               agg_lo, agg_hi, eidx_v, didx_v, rows0_v, rows1_v,
               sem0, sem1, acc_sh):
    c = lax.axis_index("c")
    s = lax.axis_index("s")
    r_per_w = n_dst // NSUB
    rchunk = min(r_per_w, EC)
    rows = (rows0_v, rows1_v)
    sems = (sem0, sem1)

    pltpu.sync_copy(zeros_h.at[pl.ds(0, rchunk)], rows0_v.at[pl.ds(0, rchunk)])
    for r in range(r_per_w // rchunk):
        pltpu.sync_copy(rows0_v.at[pl.ds(0, rchunk)],
                        acc_sh.at[pl.ds(s * r_per_w + r * rchunk, rchunk)])

    pltpu.sync_copy(src2.at[pl.ds(s * nch, nch)], eidx_v)
    pltpu.sync_copy(dst2.at[pl.ds(s * nch, nch)], didx_v)
    plsc.subcore_barrier()

    def start_gather(k, b):
        @pl.when(c == 0)
        def _():
            pltpu.async_copy(lo_t.at[eidx_v.at[k]], rows[b], sems[b])

        @pl.when(c == 1)
        def _():
            pltpu.async_copy(hi_t.at[eidx_v.at[k]], rows[b], sems[b])

    def wait_gather(k, b):
        pltpu.make_async_copy(lo_t.at[eidx_v.at[k]], rows[b], sems[b]).wait()

    start_gather(0, 0)

    def body(g, carry):
        k0 = 2 * g
        start_gather(k0 + 1, 1)
        wait_gather(k0, 0)
        pltpu.sync_copy(rows[0], acc_sh.at[didx_v.at[k0]], add=True)

        @pl.when(k0 + 2 < nch)
        def _():
            start_gather(k0 + 2, 0)

        wait_gather(k0 + 1, 1)
        pltpu.sync_copy(rows[1], acc_sh.at[didx_v.at[k0 + 1]], add=True)
        return carry

    lax.fori_loop(0, nch // 2, body, 0)
    plsc.subcore_barrier()

    def copy_out(agg):
        for r in range(r_per_w // rchunk):
            off = s * r_per_w + r * rchunk
            pltpu.sync_copy(acc_sh.at[pl.ds(off, rchunk)],
                            rows0_v.at[pl.ds(0, rchunk)])
            pltpu.sync_copy(rows0_v.at[pl.ds(0, rchunk)],
                            agg.at[pl.ds(off, rchunk)])

    @pl.when(c == 0)
    def _():
        copy_out(agg_lo)

    @pl.when(c == 1)
    def _():
        copy_out(agg_hi)


def _edge_conv(lo_t, hi_t, src, dst, n_dst, n_edges):
    nch = n_edges // NSUB // EC
    zeros_h = jnp.zeros((EC, DHALF), jnp.float32)
    f = pl.kernel(
        functools.partial(_conv_body, n_dst, nch),
        out_type=[jax.ShapeDtypeStruct((n_dst, DHALF), jnp.float32),
                  jax.ShapeDtypeStruct((n_dst, DHALF), jnp.float32)],
        mesh=_sc_mesh(),
        scratch_types=[pltpu.VMEM((nch, EC), jnp.int32),
                       pltpu.VMEM((nch, EC), jnp.int32),
                       pltpu.VMEM((EC, DHALF), jnp.float32),
                       pltpu.VMEM((EC, DHALF), jnp.float32),
                       pltpu.SemaphoreType.DMA,
                       pltpu.SemaphoreType.DMA,
                       pltpu.VMEM_SHARED((n_dst, DHALF), jnp.float32)],
    )
    return f(lo_t, hi_t, src.reshape(-1, EC), dst.reshape(-1, EC), zeros_h)


def _mid_body(alo_ref, ahi_ref, cin_ref, w_ref, b_ref, g_ref, be_ref,
              p1_ref, j1_ref, cout_ref, lo_ref, hi_ref):
    rdi = lax.rsqrt(jnp.maximum(cin_ref[...], 1.0))
    h = (jnp.dot(alo_ref[...] * rdi, w_ref[:DHALF, :],
                 preferred_element_type=jnp.float32)
         + jnp.dot(ahi_ref[...] * rdi, w_ref[DHALF:, :],
                   preferred_element_type=jnp.float32)
         + b_ref[...])
    mean = jnp.mean(h, axis=0, keepdims=True)
    var = jnp.mean((h - mean) ** 2, axis=0, keepdims=True)
    hn = (h - mean) * lax.rsqrt(var + 1e-5) * g_ref[...] + be_ref[...]
    hr = jnp.maximum(hn, 0.0)
    head = hr[:B, :]

    parts = [head]
    jj = lax.broadcasted_iota(jnp.int32, (B, B), 1)
    for blk in range(P1_ROWS // B):
        jb = j1_ref[blk]
        oh = (jb == jj).astype(jnp.float32)
        o1 = jnp.dot(oh, head, preferred_element_type=jnp.float32)
        pb = p1_ref[blk * B:(blk + 1) * B, :]
        parts.append(jnp.where(jb >= 0, o1, pb))
    full = jnp.concatenate(parts, axis=0)
    hs = full * lax.rsqrt(jnp.maximum(cout_ref[...], 1.0))
    lo_ref[...] = hs[:, :DHALF]
    hi_ref[...] = hs[:, DHALF:]


def _mid_layer(agg_lo, agg_hi, cin0, W0, b0, gamma, beta, p1, j13, cout1):
    return pl.pallas_call(
        _mid_body,
        out_shape=[jax.ShapeDtypeStruct((N1, DHALF), jnp.float32),
                   jax.ShapeDtypeStruct((N1, DHALF), jnp.float32)],
    )(agg_lo, agg_hi, cin0.reshape(N1, 1), W0, b0.reshape(1, D),
      gamma.reshape(1, D), beta.reshape(1, D), p1, j13,
      cout1.reshape(N1, 1))


def _out_body(alo_ref, ahi_ref, cin_ref, w_ref, b_ref, out_ref):
    rdi = lax.rsqrt(jnp.maximum(cin_ref[...], 1.0))
    z = (jnp.dot(alo_ref[...] * rdi, w_ref[:DHALF, :],
                 preferred_element_type=jnp.float32)
         + jnp.dot(ahi_ref[...] * rdi, w_ref[DHALF:, :],
                   preferred_element_type=jnp.float32)
         + b_ref[...])
    m = jnp.max(z, axis=1, keepdims=True)
    lse = m + jnp.log(jnp.sum(jnp.exp(z - m), axis=1, keepdims=True))
    out_ref[...] = z - lse


def _out_layer(agg_lo, agg_hi, cin1, W1, b1):
    return pl.pallas_call(
        _out_body,
        out_shape=jax.ShapeDtypeStruct((B, NCLS), jnp.float32),
    )(agg_lo, agg_hi, cin1.reshape(B, 1), W1, b1.reshape(1, NCLS))


def kernel(x, src0, dst0, src1, dst1, nid0, nid1, hist0, hist1,
           W0, b0, gamma, beta, W1, b1, batch_size):
    del batch_size
    i32 = jnp.int32
    src0, dst0 = src0.astype(i32), dst0.astype(i32)
    src1, dst1 = src1.astype(i32), dst1.astype(i32)
    nid0, nid1 = nid0.astype(i32), nid1.astype(i32)

    j0raw3, j0clip3 = _match_maps(nid0[B:].reshape(-1, 1, B),
                                  nid0[:B].reshape(1, B), P0_ROWS // B)
    j1raw3, j1clip3 = _match_maps(nid1[B:].reshape(-1, 1, B),
                                  nid1[:B].reshape(1, B), P1_ROWS // B)

    p0, o0, p1, c_out0, c_in0, c_out1, c_in1 = _pulls_counts(
        hist0, x, hist1, nid0[B:], j0clip3.reshape(-1), nid1[B:],
        src0, dst0, src1, dst1)

    xs_lo, xs_hi = _assemble_xs(
        x[:B].reshape(1, B, D), p0.reshape(-1, B, D), o0.reshape(-1, B, D),
        j0raw3.reshape(-1, B, 1), c_out0.reshape(-1, B, 1))

    agg_lo, agg_hi = _edge_conv(xs_lo, xs_hi, src0, dst0, N1, E0)

    hs_lo, hs_hi = _mid_layer(agg_lo, agg_hi, c_in0, W0, b0, gamma, beta,
                              p1, j1raw3.reshape(-1, B, 1), c_out1)

    agg1_lo, agg1_hi = _edge_conv(hs_lo, hs_hi, src1, dst1, B, E1)

    return _out_layer(agg1_lo, agg1_hi, c_in1, W1, b1)
